# Initial kernel scaffold; baseline (speedup 1.0000x reference)
import jax
import jax.numpy as jnp
from jax.experimental import pallas as pl


def kernel(x, emb_table, W, b):
    raise NotImplementedError("write your pallas kernel here")


# speedup vs baseline: 5.7413x; 5.7413x over previous
import functools

import jax
import jax.numpy as jnp
from jax import lax
from jax.experimental import pallas as pl
from jax.experimental.pallas import tpu as pltpu
from jax.experimental.pallas import tpu_sc as plsc

B = 16384
L = 4
VOCAB = 6
EMB = 100
OUT = 2
NSET = 1 << VOCAB

NC = 2
NS = 16
LANES = 16
NW = NC * NS
RPW = B // NW
GROUPS = RPW // LANES

_FMIN = -3.4028235e38


def _lut_body(table_ref, w_ref, b_ref, out_ref):
    s = lax.broadcasted_iota(jnp.int32, (NSET, 128), 0)
    pooled = jnp.full((NSET, 128), _FMIN, dtype=jnp.float32)
    for v in range(VOCAB):
        row = table_ref[v, :][None, :]
        sel = ((s >> v) & 1) == 1
        pooled = jnp.where(sel, jnp.maximum(pooled, row), pooled)
    pooled = jnp.where(s == 0, jnp.float32(0.0), pooled)
    logits = jnp.dot(pooled, w_ref[:, :], preferred_element_type=jnp.float32)
    lane = lax.broadcasted_iota(jnp.int32, (NSET, 128), 1)
    logits = jnp.where(lane < OUT, logits + b_ref[0, :][None, :], _FMIN)
    m = jnp.max(logits, axis=1, keepdims=True)
    e = jnp.exp(logits - m)
    out_ref[:, :] = e / jnp.sum(e, axis=1, keepdims=True)


_lut_call = pl.pallas_call(
    _lut_body,
    out_shape=jax.ShapeDtypeStruct((NSET, 128), jnp.float32),
)


def _sc_body(x_hbm, lut_hbm, out_hbm, xv, lutv, outv):
    wid = lax.axis_index("s") * NC + lax.axis_index("c")
    base = wid * RPW
    pltpu.sync_copy(x_hbm.at[pl.ds(base * L, RPW * L)], xv)
    pltpu.sync_copy(lut_hbm, lutv)
    lane = lax.iota(jnp.int32, LANES)
    one = jnp.full((LANES,), 1, jnp.int32)
    for i in range(GROUPS):
        r4 = (i * LANES * L) + lane * L
        x0 = plsc.load_gather(xv, [r4])
        x1 = plsc.load_gather(xv, [r4 + 1])
        x2 = plsc.load_gather(xv, [r4 + 2])
        x3 = plsc.load_gather(xv, [r4 + 3])
        m = (one << x0) | (one << x1) | (one << x2) | (one << x3)
        p0 = plsc.load_gather(lutv, [m * OUT])
        p1 = plsc.load_gather(lutv, [m * OUT + 1])
        o = (i * LANES * OUT) + lane * OUT
        plsc.store_scatter(outv, [o], p0)
        plsc.store_scatter(outv, [o + 1], p1)
    pltpu.sync_copy(outv, out_hbm.at[pl.ds(base * OUT, RPW * OUT)])


@functools.lru_cache(maxsize=None)
def _make_sc_call():
    return pl.kernel(
        _sc_body,
        out_type=jax.ShapeDtypeStruct((B * OUT,), jnp.float32),
        mesh=plsc.VectorSubcoreMesh(core_axis_name="c", subcore_axis_name="s"),
        compiler_params=pltpu.CompilerParams(needs_layout_passes=False),
        scratch_types=[
            pltpu.VMEM((RPW * L,), jnp.int32),
            pltpu.VMEM((NSET * OUT,), jnp.float32),
            pltpu.VMEM((RPW * OUT,), jnp.float32),
        ],
    )


@jax.jit
def kernel(x, emb_table, W, b):
    x = x.astype(jnp.int32)
    tpad = jnp.zeros((8, 128), jnp.float32).at[:VOCAB, :EMB].set(emb_table)
    wpad = jnp.zeros((128, 128), jnp.float32).at[:EMB, :OUT].set(W)
    bpad = jnp.zeros((1, 128), jnp.float32).at[0, :OUT].set(b)
    lut = _lut_call(tpad, wpad, bpad)[:, :OUT].reshape(NSET * OUT)
    out = _make_sc_call()(x.reshape(B * L), lut)
    return out.reshape(B, OUT)


# speedup vs baseline: 12.4456x; 2.1677x over previous
import functools

import jax
import jax.numpy as jnp
from jax import lax
from jax.experimental import pallas as pl
from jax.experimental.pallas import tpu as pltpu
from jax.experimental.pallas import tpu_sc as plsc

B = 16384
L = 4
VOCAB = 6
EMB = 100
OUT = 2
NSET = 1 << VOCAB

NC = 2
NS = 16
LANES = 16
NW = NC * NS
RPW = B // NW
GROUPS = RPW // LANES

_FMIN = -3.4028235e38


def _lut_body(table_ref, w_ref, b_ref, out_ref):
    s = lax.broadcasted_iota(jnp.int32, (NSET, 128), 0)
    pooled = jnp.full((NSET, 128), _FMIN, dtype=jnp.float32)
    for v in range(VOCAB):
        row = table_ref[v, :][None, :]
        sel = ((s >> v) & 1) == 1
        pooled = jnp.where(sel, jnp.maximum(pooled, row), pooled)
    pooled = jnp.where(s == 0, jnp.float32(0.0), pooled)
    logits = jnp.dot(pooled, w_ref[:, :], preferred_element_type=jnp.float32)
    lane = lax.broadcasted_iota(jnp.int32, (NSET, 128), 1)
    logits = jnp.where(lane < OUT, logits + b_ref[0, :][None, :], _FMIN)
    m = jnp.max(logits, axis=1, keepdims=True)
    e = jnp.exp(logits - m)
    out_ref[:, :] = e / jnp.sum(e, axis=1, keepdims=True)


_lut_call = pl.pallas_call(
    _lut_body,
    out_shape=jax.ShapeDtypeStruct((NSET, 128), jnp.float32),
)


def _sc_body(x_hbm, lut_hbm, out_hbm, xv, lutv, outv):
    wid = lax.axis_index("s") * NC + lax.axis_index("c")
    base = wid * RPW
    pltpu.sync_copy(x_hbm.at[pl.ds(base * L, RPW * L)], xv)
    pltpu.sync_copy(lut_hbm, lutv)
    one = jnp.full((LANES,), 1, jnp.int32)
    for i in range(GROUPS):
        xoff = (i // 8) * (128 * L) + (i % 8) * LANES
        x0 = xv[pl.ds(xoff, LANES)]
        x1 = xv[pl.ds(xoff + 128, LANES)]
        x2 = xv[pl.ds(xoff + 256, LANES)]
        x3 = xv[pl.ds(xoff + 384, LANES)]
        m = (one << x0) | (one << x1) | (one << x2) | (one << x3)
        p0 = plsc.load_gather(lutv, [m * 128])
        p1 = plsc.load_gather(lutv, [m * 128 + 1])
        ooff = (i // 8) * (128 * OUT) + (i % 8) * LANES
        outv[pl.ds(ooff, LANES)] = p0
        outv[pl.ds(ooff + 128, LANES)] = p1
    pltpu.sync_copy(outv, out_hbm.at[pl.ds(base * OUT, RPW * OUT)])


@functools.lru_cache(maxsize=None)
def _make_sc_call():
    return pl.kernel(
        _sc_body,
        out_type=jax.ShapeDtypeStruct((B * OUT,), jnp.float32),
        mesh=plsc.VectorSubcoreMesh(core_axis_name="c", subcore_axis_name="s"),
        compiler_params=pltpu.CompilerParams(needs_layout_passes=False),
        scratch_types=[
            pltpu.VMEM((RPW * L,), jnp.int32),
            pltpu.VMEM((NSET * 128,), jnp.float32),
            pltpu.VMEM((RPW * OUT,), jnp.float32),
        ],
    )


@jax.jit
def kernel(x, emb_table, W, b):
    x = x.astype(jnp.int32)
    tpad = jnp.zeros((8, 128), jnp.float32).at[:VOCAB, :EMB].set(emb_table)
    wpad = jnp.zeros((128, 128), jnp.float32).at[:EMB, :OUT].set(W)
    bpad = jnp.zeros((1, 128), jnp.float32).at[0, :OUT].set(b)
    lut = _lut_call(tpad, wpad, bpad).reshape(NSET * 128)
    xp = x.reshape(B // 128, 128, L).transpose(0, 2, 1).reshape(B * L)
    out = _make_sc_call()(xp, lut)
    return out.reshape(B // 128, OUT, 128).transpose(0, 2, 1).reshape(B, OUT)
